# baseline (speedup 1.0000x reference)
import jax
import jax.numpy as jnp
from jax.experimental import pallas as pl
from jax.experimental.pallas import tpu as pltpu

_S = 16384
_Q = 4096
_D = 128
_C = 64
_K = 5

_BQ = 256
_CH = 1024


def _ncm_body(scls_ref, qcls_ref, q_ref, s_ref, ssq_ref, out_ref,
              scores_ref, bias_ref, adj_ref):
    i = pl.program_id(0)

    q = q_ref[...]
    qc = qcls_ref[0, 0, :][:, None]

    @pl.when(i == 0)
    def _build_bias():
        for c in range(_S // _CH):
            sl = pl.ds(c * _CH, _CH)
            cls_row = jax.lax.broadcasted_iota(jnp.int32, (_C, _CH), 0)
            bias_ref[:, sl] = jnp.where(scls_ref[:, sl] == cls_row, 0.0, 1e30)

    iota_c = jax.lax.broadcasted_iota(jnp.int32, (_BQ, _C), 1)
    onehot = jnp.where(iota_c == qc, 1.0, 0.0).astype(jnp.float32)

    q2 = q + q
    accm = jnp.full((_BQ, _CH), jnp.inf, jnp.float32)
    for c in range(_S // _CH):
        sl = pl.ds(c * _CH, _CH)
        dots2 = jax.lax.dot_general(
            q2, s_ref[sl, :], (((1,), (1,)), ((), ())),
            preferred_element_type=jnp.float32)
        gb = jax.lax.dot_general(
            onehot, bias_ref[:, sl], (((1,), (0,)), ((), ())),
            preferred_element_type=jnp.float32)
        sc = ssq_ref[:, sl] - dots2
        scores_ref[:, sl] = sc
        accm = jnp.minimum(accm, sc + gb)
    m = jnp.min(accm, axis=1, keepdims=True)

    acc2 = jnp.zeros((_BQ, _CH), jnp.float32)
    one = jnp.ones((_BQ, _CH), jnp.float32)
    eqw = jnp.full((_BQ, _CH), 32768.0, jnp.float32)
    zero = jnp.zeros((_BQ, _CH), jnp.float32)
    for c in range(_S // _CH):
        sl = pl.ds(c * _CH, _CH)
        sc = scores_ref[:, sl]
        acc2 += jnp.where(sc < m, one, jnp.where(sc == m, eqw, zero))
    comb = jnp.sum(acc2, axis=1, keepdims=True)
    cnt_eq = jnp.floor(comb * (1.0 / 32768.0))
    cnt_lt = comb - cnt_eq * 32768.0

    amb = (cnt_lt < _K) & (cnt_lt + cnt_eq > _K)
    adj_ref[...] = jnp.zeros((_BQ, 1), jnp.float32)

    @pl.when(jnp.any(amb))
    def _resolve_ties():
        idx_m = jnp.full((_BQ, 1), _S, jnp.int32)
        for c in range(_S // _CH):
            sl = pl.ds(c * _CH, _CH)
            sc = scores_ref[:, sl]
            hit = (sc == m) & (scls_ref[:, sl] == qc)
            iota = jax.lax.broadcasted_iota(jnp.int32, (_BQ, _CH), 1) + c * _CH
            idx_m = jnp.minimum(
                idx_m, jnp.min(jnp.where(hit, iota, _S), axis=1, keepdims=True))
        eq_before = jnp.zeros((_BQ, 1), jnp.float32)
        for c in range(_S // _CH):
            sl = pl.ds(c * _CH, _CH)
            sc = scores_ref[:, sl]
            iota = jax.lax.broadcasted_iota(jnp.int32, (_BQ, _CH), 1) + c * _CH
            eq_before += jnp.sum(((sc == m) & (iota < idx_m)).astype(jnp.float32),
                                 axis=1, keepdims=True)
        adj_ref[...] = eq_before

    cnt = cnt_lt + adj_ref[...]
    part = jnp.sum((cnt < _K).astype(jnp.float32)) * (1.0 / _Q)

    @pl.when(i == 0)
    def _init_out():
        out_ref[...] = jnp.zeros((1, 1), jnp.float32)

    out_ref[...] = out_ref[...] + part


def kernel(support_features, query_features, support_labels, query_labels):
    scls = support_labels[:, 0].reshape(1, _S)
    qcls = query_labels[:, 0].reshape(_Q // _BQ, 1, _BQ)
    ssq = jnp.sum(support_features * support_features, axis=1).reshape(1, _S)

    acc = pl.pallas_call(
        _ncm_body,
        grid=(_Q // _BQ,),
        in_specs=[
            pl.BlockSpec((1, _S), lambda i: (0, 0)),
            pl.BlockSpec((1, 1, _BQ), lambda i: (i, 0, 0)),
            pl.BlockSpec((_BQ, _D), lambda i: (i, 0)),
            pl.BlockSpec((_S, _D), lambda i: (0, 0)),
            pl.BlockSpec((1, _S), lambda i: (0, 0)),
        ],
        out_specs=pl.BlockSpec((1, 1), lambda i: (0, 0)),
        out_shape=jax.ShapeDtypeStruct((1, 1), jnp.float32),
        scratch_shapes=[
            pltpu.VMEM((_BQ, _S), jnp.float32),
            pltpu.VMEM((_C, _S), jnp.float32),
            pltpu.VMEM((_BQ, 1), jnp.float32),
        ],
        compiler_params=pltpu.CompilerParams(
            dimension_semantics=("arbitrary",),
            vmem_limit_bytes=60 * 1024 * 1024,
        ),
    )(scls, qcls, query_features, support_features, ssq)
    return acc[0, 0]


# speedup vs baseline: 1.3455x; 1.3455x over previous
import jax
import jax.numpy as jnp
from jax.experimental import pallas as pl
from jax.experimental.pallas import tpu as pltpu

_S = 16384
_Q = 4096
_D = 128
_K = 5

_BQ = 256
_CH = 512


def _ncm_body(scls_ref, qcls_ref, q_ref, s_ref, ssq_ref, out_ref,
              scores_ref, adj_ref):
    i = pl.program_id(0)

    q = q_ref[...]
    qc = qcls_ref[0, 0, :][:, None]

    q2 = q + q
    accm = jnp.full((_BQ, _CH), jnp.inf, jnp.float32)
    for c in range(_S // _CH):
        sl = pl.ds(c * _CH, _CH)
        dots2 = jax.lax.dot_general(
            q2, s_ref[sl, :], (((1,), (1,)), ((), ())),
            preferred_element_type=jnp.float32)
        sc = ssq_ref[:, sl] - dots2
        scores_ref[:, sl] = sc
        accm = jnp.minimum(accm, jnp.where(scls_ref[:, sl] == qc, sc, jnp.inf))
    m = jnp.min(accm, axis=1, keepdims=True)

    acc2 = jnp.zeros((_BQ, _CH), jnp.float32)
    one = jnp.ones((_BQ, _CH), jnp.float32)
    eqw = jnp.full((_BQ, _CH), 32768.0, jnp.float32)
    zero = jnp.zeros((_BQ, _CH), jnp.float32)
    for c in range(_S // _CH):
        sl = pl.ds(c * _CH, _CH)
        sc = scores_ref[:, sl]
        acc2 += jnp.where(sc < m, one, jnp.where(sc == m, eqw, zero))
    comb = jnp.sum(acc2, axis=1, keepdims=True)
    cnt_eq = jnp.floor(comb * (1.0 / 32768.0))
    cnt_lt = comb - cnt_eq * 32768.0

    amb = (cnt_lt < _K) & (cnt_lt + cnt_eq > _K)
    adj_ref[...] = jnp.zeros((_BQ, 1), jnp.float32)

    @pl.when(jnp.any(amb))
    def _resolve_ties():
        idx_m = jnp.full((_BQ, 1), _S, jnp.int32)
        for c in range(_S // _CH):
            sl = pl.ds(c * _CH, _CH)
            sc = scores_ref[:, sl]
            hit = (sc == m) & (scls_ref[:, sl] == qc)
            iota = jax.lax.broadcasted_iota(jnp.int32, (_BQ, _CH), 1) + c * _CH
            idx_m = jnp.minimum(
                idx_m, jnp.min(jnp.where(hit, iota, _S), axis=1, keepdims=True))
        eq_before = jnp.zeros((_BQ, 1), jnp.float32)
        for c in range(_S // _CH):
            sl = pl.ds(c * _CH, _CH)
            sc = scores_ref[:, sl]
            iota = jax.lax.broadcasted_iota(jnp.int32, (_BQ, _CH), 1) + c * _CH
            eq_before += jnp.sum(((sc == m) & (iota < idx_m)).astype(jnp.float32),
                                 axis=1, keepdims=True)
        adj_ref[...] = eq_before

    cnt = cnt_lt + adj_ref[...]
    part = jnp.sum((cnt < _K).astype(jnp.float32)) * (1.0 / _Q)

    @pl.when(i == 0)
    def _init_out():
        out_ref[...] = jnp.zeros((1, 1), jnp.float32)

    out_ref[...] = out_ref[...] + part


def kernel(support_features, query_features, support_labels, query_labels):
    scls = support_labels[:, 0].reshape(1, _S)
    qcls = query_labels[:, 0].reshape(_Q // _BQ, 1, _BQ)
    ssq = jnp.sum(support_features * support_features, axis=1).reshape(1, _S)

    acc = pl.pallas_call(
        _ncm_body,
        grid=(_Q // _BQ,),
        in_specs=[
            pl.BlockSpec((1, _S), lambda i: (0, 0)),
            pl.BlockSpec((1, 1, _BQ), lambda i: (i, 0, 0)),
            pl.BlockSpec((_BQ, _D), lambda i: (i, 0)),
            pl.BlockSpec((_S, _D), lambda i: (0, 0)),
            pl.BlockSpec((1, _S), lambda i: (0, 0)),
        ],
        out_specs=pl.BlockSpec((1, 1), lambda i: (0, 0)),
        out_shape=jax.ShapeDtypeStruct((1, 1), jnp.float32),
        scratch_shapes=[
            pltpu.VMEM((_BQ, _S), jnp.float32),
            pltpu.VMEM((_BQ, 1), jnp.float32),
        ],
        compiler_params=pltpu.CompilerParams(
            dimension_semantics=("arbitrary",),
            vmem_limit_bytes=60 * 1024 * 1024,
        ),
    )(scls, qcls, query_features, support_features, ssq)
    return acc[0, 0]


# speedup vs baseline: 1.3505x; 1.0037x over previous
import jax
import jax.numpy as jnp
from jax.experimental import pallas as pl
from jax.experimental.pallas import tpu as pltpu

_S = 16384
_Q = 4096
_D = 128
_K = 5

_BQ = 256
_CH = 1024


def _ncm_body(scls_ref, qcls_ref, q_ref, s_ref, ssq_ref, out_ref,
              scores_ref, adj_ref):
    i = pl.program_id(0)

    q = q_ref[...]
    qc = qcls_ref[0, 0, :][:, None]

    q2 = q + q
    accm = jnp.full((_BQ, _CH), jnp.inf, jnp.float32)
    for c in range(_S // _CH):
        sl = pl.ds(c * _CH, _CH)
        dots2 = jax.lax.dot_general(
            q2, s_ref[sl, :], (((1,), (1,)), ((), ())),
            preferred_element_type=jnp.float32)
        sc = ssq_ref[:, sl] - dots2
        scores_ref[:, sl] = sc
        accm = jnp.minimum(accm, jnp.where(scls_ref[:, sl] == qc, sc, jnp.inf))
    m = jnp.min(accm, axis=1, keepdims=True)

    acc2 = jnp.zeros((_BQ, _CH), jnp.float32)
    one = jnp.ones((_BQ, _CH), jnp.float32)
    eqw = jnp.full((_BQ, _CH), 32768.0, jnp.float32)
    zero = jnp.zeros((_BQ, _CH), jnp.float32)
    for c in range(_S // _CH):
        sl = pl.ds(c * _CH, _CH)
        sc = scores_ref[:, sl]
        acc2 += jnp.where(sc < m, one, jnp.where(sc == m, eqw, zero))
    comb = jnp.sum(acc2, axis=1, keepdims=True)
    cnt_eq = jnp.floor(comb * (1.0 / 32768.0))
    cnt_lt = comb - cnt_eq * 32768.0

    amb = (cnt_lt < _K) & (cnt_lt + cnt_eq > _K)
    adj_ref[...] = jnp.zeros((_BQ, 1), jnp.float32)

    @pl.when(jnp.any(amb))
    def _resolve_ties():
        idx_m = jnp.full((_BQ, 1), _S, jnp.int32)
        for c in range(_S // _CH):
            sl = pl.ds(c * _CH, _CH)
            sc = scores_ref[:, sl]
            hit = (sc == m) & (scls_ref[:, sl] == qc)
            iota = jax.lax.broadcasted_iota(jnp.int32, (_BQ, _CH), 1) + c * _CH
            idx_m = jnp.minimum(
                idx_m, jnp.min(jnp.where(hit, iota, _S), axis=1, keepdims=True))
        eq_before = jnp.zeros((_BQ, 1), jnp.float32)
        for c in range(_S // _CH):
            sl = pl.ds(c * _CH, _CH)
            sc = scores_ref[:, sl]
            iota = jax.lax.broadcasted_iota(jnp.int32, (_BQ, _CH), 1) + c * _CH
            eq_before += jnp.sum(((sc == m) & (iota < idx_m)).astype(jnp.float32),
                                 axis=1, keepdims=True)
        adj_ref[...] = eq_before

    cnt = cnt_lt + adj_ref[...]
    part = jnp.sum((cnt < _K).astype(jnp.float32)) * (1.0 / _Q)

    @pl.when(i == 0)
    def _init_out():
        out_ref[...] = jnp.zeros((1, 1), jnp.float32)

    out_ref[...] = out_ref[...] + part


def kernel(support_features, query_features, support_labels, query_labels):
    scls = support_labels[:, 0].reshape(1, _S)
    qcls = query_labels[:, 0].reshape(_Q // _BQ, 1, _BQ)
    ssq = jnp.sum(support_features * support_features, axis=1).reshape(1, _S)

    acc = pl.pallas_call(
        _ncm_body,
        grid=(_Q // _BQ,),
        in_specs=[
            pl.BlockSpec((1, _S), lambda i: (0, 0)),
            pl.BlockSpec((1, 1, _BQ), lambda i: (i, 0, 0)),
            pl.BlockSpec((_BQ, _D), lambda i: (i, 0)),
            pl.BlockSpec((_S, _D), lambda i: (0, 0)),
            pl.BlockSpec((1, _S), lambda i: (0, 0)),
        ],
        out_specs=pl.BlockSpec((1, 1), lambda i: (0, 0)),
        out_shape=jax.ShapeDtypeStruct((1, 1), jnp.float32),
        scratch_shapes=[
            pltpu.VMEM((_BQ, _S), jnp.float32),
            pltpu.VMEM((_BQ, 1), jnp.float32),
        ],
        compiler_params=pltpu.CompilerParams(
            dimension_semantics=("arbitrary",),
            vmem_limit_bytes=60 * 1024 * 1024,
        ),
    )(scls, qcls, query_features, support_features, ssq)
    return acc[0, 0]
